# Initial kernel scaffold; baseline (speedup 1.0000x reference)
import jax
import jax.numpy as jnp
from jax.experimental import pallas as pl


def kernel(x_static, x_dynamic, edge_index, edge_weight, bn0_w, bn0_b, W1, b1, bn1_w, bn1_b, W2, b2, bn2_w, bn2_b, lin_W, lin_b):
    raise NotImplementedError("write your pallas kernel here")


# speedup vs baseline: 2.4349x; 2.4349x over previous
import functools

import jax
import jax.numpy as jnp
from jax import lax
from jax.experimental import pallas as pl
from jax.experimental.pallas import tpu as pltpu
from jax.experimental.pallas import tpu_sc as plsc

N = 10000
NP = 10240
E = 320000
CH = 128
NCHUNK = 2560
EP = NCHUNK * CH
NC = 2
NS = 16
HID = 128
COMB = 128
BN_SCALE = 1.0 / (1.0 + 1e-5) ** 0.5

_i32 = jnp.int32
_f32 = jnp.float32


def _sc_mesh():
    return plsc.VectorSubcoreMesh(
        core_axis_name="c", subcore_axis_name="s", num_cores=NC, num_subcores=NS
    )


def _iota16():
    return lax.iota(_i32, 16)


def _clean_ew16(v):
    v = jnp.where(v != v, 0.0, v)
    return jnp.clip(v, 0.0, 1e6)


def _rsqrt16(x):
    i = lax.bitcast_convert_type(x, _i32)
    y = lax.bitcast_convert_type(jnp.int32(0x5F3759DF) - (i >> 1), _f32)
    for _ in range(3):
        y = y * (1.5 - 0.5 * x * y * y)
    return y


_DEG_CPT = NCHUNK // NS
_DEG_NPT = NP // (NC * NS)
_ROWS_PT = NP // NS


def _sc_deg_body(dst_hbm, ew_hbm, zv_hbm, deg16_hbm, dinv_hbm,
                 dstv, ewv, bounce, dinvbuf, b16, degsp):
    cid = lax.axis_index("c")
    sid = lax.axis_index("s")
    base = sid * _DEG_CPT
    pltpu.sync_copy(dst_hbm.at[pl.ds(base, _DEG_CPT)], dstv)
    pltpu.sync_copy(ew_hbm.at[pl.ds(base, _DEG_CPT)], ewv)
    pltpu.sync_copy(zv_hbm, degsp.at[pl.ds(sid * _ROWS_PT, _ROWS_PT)])
    plsc.subcore_barrier()

    it16 = _iota16()

    def clean_body(i, carry):
        c = i >> 3
        g = i & 7
        ewv[c, pl.ds(g * 16, 16)] = _clean_ew16(ewv[c, pl.ds(g * 16, 16)])
        return carry

    lax.fori_loop(0, _DEG_CPT * 8, clean_body, 0)

    def chunk_body(c, carry):
        pltpu.sync_copy(ewv.at[c], degsp.at[dstv.at[c]], add=True)
        return carry

    lax.fori_loop(0, _DEG_CPT, chunk_body, 0)
    plsc.subcore_barrier()

    node0 = cid * (NP // NC) + sid * _DEG_NPT
    pltpu.sync_copy(degsp.at[pl.ds(node0, _DEG_NPT)], bounce)

    def dinv_body(k, carry):
        d16 = bounce[pl.ds(k * 16, 16)]
        dinvbuf[pl.ds(k * 16, 16)] = _rsqrt16(1.0 + d16)
        plsc.store_scatter(b16, [(it16 + k * 16) * 16], d16)
        return carry

    lax.fori_loop(0, _DEG_NPT // 16, dinv_body, 0)
    pltpu.sync_copy(b16, deg16_hbm.at[pl.ds(node0 * 16, _DEG_NPT * 16)])
    pltpu.sync_copy(dinvbuf, dinv_hbm.at[pl.ds(node0, _DEG_NPT)])


def _sc_deg(dst2, ew2, zv):
    kfn = pl.kernel(
        _sc_deg_body,
        out_type=(
            jax.ShapeDtypeStruct((NP * 16,), _f32),
            jax.ShapeDtypeStruct((NP,), _f32),
        ),
        mesh=_sc_mesh(),
        compiler_params=pltpu.CompilerParams(needs_layout_passes=False),
        scratch_types=[
            pltpu.VMEM((_DEG_CPT, CH), _i32),
            pltpu.VMEM((_DEG_CPT, CH), _f32),
            pltpu.VMEM((_DEG_NPT,), _f32),
            pltpu.VMEM((_DEG_NPT,), _f32),
            pltpu.VMEM((_DEG_NPT * 16,), _f32),
            pltpu.VMEM_SHARED((NP,), _f32),
        ],
    )
    return kfn(dst2, ew2, zv)


_CPT = NCHUNK // (NC * NS)


_B8 = 8


def _sc_layer_body(src_hbm, dst_hbm, ew_hbm, dinv_hbm, h_hbm, z128_hbm,
                   out_hbm, srcv, dstv, ewv, dinvv, gbuf, degsp_acc):
    cid = lax.axis_index("c")
    sid = lax.axis_index("s")
    base = cid * (NS * _CPT) + sid * _CPT
    pltpu.sync_copy(dinv_hbm, dinvv)
    for j in range(_ROWS_PT // 128):
        pltpu.sync_copy(z128_hbm, degsp_acc.at[pl.ds(sid * _ROWS_PT + j * 128, 128)])
    plsc.subcore_barrier()

    it16 = _iota16()

    def super_body(s, carry):
        hb = base + s * _B8
        pltpu.sync_copy(src_hbm.at[pl.ds(hb, _B8)], srcv)
        pltpu.sync_copy(dst_hbm.at[pl.ds(hb, _B8)], dstv)
        pltpu.sync_copy(ew_hbm.at[pl.ds(hb, _B8)], ewv)

        def chunk_body(k, carry1):
            pltpu.sync_copy(h_hbm.at[srcv.at[k]], gbuf)
            for g in range(8):
                ew16 = _clean_ew16(ewv[k, pl.ds(g * 16, 16)])
                src16 = srcv[k, pl.ds(g * 16, 16)]
                f16 = ew16 * plsc.load_gather(dinvv, [src16])
                e16 = it16 + g * 16

                def feat_body(f, carry2):
                    fv = jnp.full((16,), f, _i32)
                    v = plsc.load_gather(gbuf, [e16, fv])
                    plsc.store_scatter(gbuf, [e16, fv], v * f16)
                    return carry2

                lax.fori_loop(0, HID, feat_body, 0)
            pltpu.sync_copy(gbuf, degsp_acc.at[dstv.at[k]], add=True)
            return carry1

        lax.fori_loop(0, _B8, chunk_body, 0)
        return carry

    lax.fori_loop(0, _CPT // _B8, super_body, 0)
    plsc.subcore_barrier()

    for j in range(_ROWS_PT // 128):
        r0 = sid * _ROWS_PT + j * 128
        pltpu.sync_copy(degsp_acc.at[pl.ds(r0, 128)], gbuf)
        pltpu.sync_copy(gbuf, out_hbm.at[cid, pl.ds(r0, 128)])


def _sc_layer(src2, dst3, ew2, dinv, h, z128):
    kfn = pl.kernel(
        _sc_layer_body,
        out_type=jax.ShapeDtypeStruct((NC, NP, HID), _f32),
        mesh=_sc_mesh(),
        compiler_params=pltpu.CompilerParams(needs_layout_passes=False),
        scratch_types=[
            pltpu.VMEM((_B8, CH), _i32),
            pltpu.VMEM((_B8, CH), _i32),
            pltpu.VMEM((_B8, CH), _f32),
            pltpu.VMEM((NP,), _f32),
            pltpu.VMEM((CH, HID), _f32),
            pltpu.VMEM_SHARED((NP, HID), _f32),
        ],
    )
    return kfn(src2, dst3, ew2, dinv, h, z128)


_RB = 128
_GRID = NP // _RB


def _tc_prep_body(comb_ref, bn0w_ref, bn0b_ref, w1t_ref, h1_ref):
    x = comb_ref[...]
    x = jnp.where(jnp.isnan(x), 0.0, x)
    x = jnp.clip(x, -1e6, 1e6)
    x = x * (bn0w_ref[...] * BN_SCALE) + bn0b_ref[...]
    x = jnp.clip(x, -10.0, 10.0)
    h1_ref[...] = jnp.dot(x, w1t_ref[...], preferred_element_type=_f32)


def _tc_prep(comb_p, bn0w, bn0b, w1t):
    return pl.pallas_call(
        _tc_prep_body,
        grid=(_GRID,),
        in_specs=[
            pl.BlockSpec((_RB, COMB), lambda i: (i, 0)),
            pl.BlockSpec((1, COMB), lambda i: (0, 0)),
            pl.BlockSpec((1, COMB), lambda i: (0, 0)),
            pl.BlockSpec((COMB, HID), lambda i: (0, 0)),
        ],
        out_specs=pl.BlockSpec((_RB, HID), lambda i: (i, 0)),
        out_shape=jax.ShapeDtypeStruct((NP, HID), _f32),
    )(comb_p, bn0w, bn0b, w1t)


def _tc_post_mid_body(accp_ref, h_ref, deg16_ref, b_ref, bnw_ref, bnb_ref,
                      wt_ref, hn_ref):
    acc = accp_ref[0] + accp_ref[1]
    deg = 1.0 + deg16_ref[...][:, 0:1]
    dinv = lax.rsqrt(deg)
    o = dinv * acc + (dinv * dinv) * h_ref[...] + b_ref[...]
    o = jnp.clip(o, -10.0, 10.0)
    o = o * (bnw_ref[...] * BN_SCALE) + bnb_ref[...]
    o = jnp.clip(o, -10.0, 10.0)
    o = jnp.maximum(o, 0.0)
    hn_ref[...] = jnp.dot(o, wt_ref[...], preferred_element_type=_f32)


def _tc_post_mid(accp, h, deg16, b, bnw, bnb, wt):
    return pl.pallas_call(
        _tc_post_mid_body,
        grid=(_GRID,),
        in_specs=[
            pl.BlockSpec((NC, _RB, HID), lambda i: (0, i, 0)),
            pl.BlockSpec((_RB, HID), lambda i: (i, 0)),
            pl.BlockSpec((_RB, 16), lambda i: (i, 0)),
            pl.BlockSpec((1, HID), lambda i: (0, 0)),
            pl.BlockSpec((1, HID), lambda i: (0, 0)),
            pl.BlockSpec((1, HID), lambda i: (0, 0)),
            pl.BlockSpec((HID, HID), lambda i: (0, 0)),
        ],
        out_specs=pl.BlockSpec((_RB, HID), lambda i: (i, 0)),
        out_shape=jax.ShapeDtypeStruct((NP, HID), _f32),
    )(accp, h, deg16, b, bnw, bnb, wt)


def _tc_post_final_body(accp_ref, h_ref, deg16_ref, b_ref, bnw_ref, bnb_ref,
                        linwt_ref, linb_ref, y_ref):
    acc = accp_ref[0] + accp_ref[1]
    deg = 1.0 + deg16_ref[...][:, 0:1]
    dinv = lax.rsqrt(deg)
    o = dinv * acc + (dinv * dinv) * h_ref[...] + b_ref[...]
    o = jnp.clip(o, -10.0, 10.0)
    o = o * (bnw_ref[...] * BN_SCALE) + bnb_ref[...]
    o = jnp.clip(o, -10.0, 10.0)
    o = jnp.maximum(o, 0.0)
    y = jnp.dot(o, linwt_ref[...], preferred_element_type=_f32) + linb_ref[0, 0]
    y = jnp.clip(y, -10.0, 10.0)
    y_ref[...] = jnp.broadcast_to(y, (_RB, HID))


def _tc_post_final(accp, h, deg16, b, bnw, bnb, linwt, linb):
    return pl.pallas_call(
        _tc_post_final_body,
        grid=(_GRID,),
        in_specs=[
            pl.BlockSpec((NC, _RB, HID), lambda i: (0, i, 0)),
            pl.BlockSpec((_RB, HID), lambda i: (i, 0)),
            pl.BlockSpec((_RB, 16), lambda i: (i, 0)),
            pl.BlockSpec((1, HID), lambda i: (0, 0)),
            pl.BlockSpec((1, HID), lambda i: (0, 0)),
            pl.BlockSpec((1, HID), lambda i: (0, 0)),
            pl.BlockSpec((HID, 1), lambda i: (0, 0)),
            pl.BlockSpec((1, 1), lambda i: (0, 0)),
        ],
        out_specs=pl.BlockSpec((_RB, HID), lambda i: (i, 0)),
        out_shape=jax.ShapeDtypeStruct((NP, HID), _f32),
    )(accp, h, deg16, b, bnw, bnb, linwt, linb)


def kernel(x_static, x_dynamic, edge_index, edge_weight, bn0_w, bn0_b,
           W1, b1, bn1_w, bn1_b, W2, b2, bn2_w, bn2_b, lin_W, lin_b):
    flat = x_dynamic.reshape(N, -1)
    comb = jnp.concatenate([x_static, flat], axis=1)
    comb_p = jnp.pad(comb, ((0, NP - N), (0, 0)))

    src = edge_index[0]
    dst = edge_index[1]
    npad = EP - E
    pad_idx = (jnp.arange(npad, dtype=_i32) % (NP - N)) + N
    src_p = jnp.concatenate([src, pad_idx]).reshape(NCHUNK, CH)
    dst_p = jnp.concatenate([dst, pad_idx]).reshape(NCHUNK, CH)
    ew_p = jnp.concatenate(
        [edge_weight, jnp.zeros((npad,), _f32)]).reshape(NCHUNK, CH)

    zv = jnp.zeros((_ROWS_PT,), _f32)
    z128 = jnp.zeros((128, 128), _f32)

    deg16_flat, dinv = _sc_deg(dst_p, ew_p, zv)
    deg16 = deg16_flat.reshape(NP, 16)
    h1 = _tc_prep(comb_p, bn0_w.reshape(1, -1), bn0_b.reshape(1, -1), W1.T)
    accp1 = _sc_layer(src_p, dst_p, ew_p, dinv, h1, z128)
    h2 = _tc_post_mid(accp1, h1, deg16, b1.reshape(1, -1),
                      bn1_w.reshape(1, -1), bn1_b.reshape(1, -1), W2.T)
    accp2 = _sc_layer(src_p, dst_p, ew_p, dinv, h2, z128)
    yb = _tc_post_final(accp2, h2, deg16, b2.reshape(1, -1),
                        bn2_w.reshape(1, -1), bn2_b.reshape(1, -1),
                        lin_W.T, lin_b.reshape(1, 1))
    return yb[:N, 0]


# speedup vs baseline: 2.5712x; 1.0560x over previous
import functools

import jax
import jax.numpy as jnp
from jax import lax
from jax.experimental import pallas as pl
from jax.experimental.pallas import tpu as pltpu
from jax.experimental.pallas import tpu_sc as plsc

N = 10000
NP = 10240
E = 320000
CH = 128
NCHUNK = 2560
EP = NCHUNK * CH
NC = 2
NS = 16
HID = 128
COMB = 128
BN_SCALE = 1.0 / (1.0 + 1e-5) ** 0.5

_i32 = jnp.int32
_f32 = jnp.float32


def _sc_mesh():
    return plsc.VectorSubcoreMesh(
        core_axis_name="c", subcore_axis_name="s", num_cores=NC, num_subcores=NS
    )


def _iota16():
    return lax.iota(_i32, 16)


def _clean_ew16(v):
    v = jnp.where(v != v, 0.0, v)
    return jnp.clip(v, 0.0, 1e6)


def _rsqrt16(x):
    i = lax.bitcast_convert_type(x, _i32)
    y = lax.bitcast_convert_type(jnp.int32(0x5F3759DF) - (i >> 1), _f32)
    for _ in range(3):
        y = y * (1.5 - 0.5 * x * y * y)
    return y


_DEG_CPT = NCHUNK // NS
_DEG_NPT = NP // (NC * NS)
_ROWS_PT = NP // NS


def _sc_deg_body(dst_hbm, ew_hbm, zv_hbm, deg16_hbm, dinv_hbm,
                 dstv, ewv, bounce, dinvbuf, b16, degsp):
    cid = lax.axis_index("c")
    sid = lax.axis_index("s")
    base = sid * _DEG_CPT
    pltpu.sync_copy(dst_hbm.at[pl.ds(base, _DEG_CPT)], dstv)
    pltpu.sync_copy(ew_hbm.at[pl.ds(base, _DEG_CPT)], ewv)
    pltpu.sync_copy(zv_hbm, degsp.at[pl.ds(sid * _ROWS_PT, _ROWS_PT)])
    plsc.subcore_barrier()

    it16 = _iota16()

    def clean_body(i, carry):
        c = i >> 3
        g = i & 7
        ewv[c, pl.ds(g * 16, 16)] = _clean_ew16(ewv[c, pl.ds(g * 16, 16)])
        return carry

    lax.fori_loop(0, _DEG_CPT * 8, clean_body, 0)

    def chunk_body(c, carry):
        pltpu.sync_copy(ewv.at[c], degsp.at[dstv.at[c]], add=True)
        return carry

    lax.fori_loop(0, _DEG_CPT, chunk_body, 0)
    plsc.subcore_barrier()

    node0 = cid * (NP // NC) + sid * _DEG_NPT
    pltpu.sync_copy(degsp.at[pl.ds(node0, _DEG_NPT)], bounce)

    def dinv_body(k, carry):
        d16 = bounce[pl.ds(k * 16, 16)]
        dinvbuf[pl.ds(k * 16, 16)] = _rsqrt16(1.0 + d16)
        plsc.store_scatter(b16, [(it16 + k * 16) * 16], d16)
        return carry

    lax.fori_loop(0, _DEG_NPT // 16, dinv_body, 0)
    pltpu.sync_copy(b16, deg16_hbm.at[pl.ds(node0 * 16, _DEG_NPT * 16)])
    pltpu.sync_copy(dinvbuf, dinv_hbm.at[pl.ds(node0, _DEG_NPT)])


def _sc_deg(dst2, ew2, zv):
    kfn = pl.kernel(
        _sc_deg_body,
        out_type=(
            jax.ShapeDtypeStruct((NP * 16,), _f32),
            jax.ShapeDtypeStruct((NP,), _f32),
        ),
        mesh=_sc_mesh(),
        compiler_params=pltpu.CompilerParams(needs_layout_passes=False),
        scratch_types=[
            pltpu.VMEM((_DEG_CPT, CH), _i32),
            pltpu.VMEM((_DEG_CPT, CH), _f32),
            pltpu.VMEM((_DEG_NPT,), _f32),
            pltpu.VMEM((_DEG_NPT,), _f32),
            pltpu.VMEM((_DEG_NPT * 16,), _f32),
            pltpu.VMEM_SHARED((NP,), _f32),
        ],
    )
    return kfn(dst2, ew2, zv)


_CPT = NCHUNK // (NC * NS)


_B8 = 8


def _sc_layer_body(src_hbm, dst_hbm, ew_hbm, dinv_hbm, h_hbm, z128_hbm,
                   out_hbm, srcv, dstv, ewv, dinvv, g0, g1,
                   gs0, gs1, ss0, ss1, degsp_acc):
    cid = lax.axis_index("c")
    sid = lax.axis_index("s")
    base = cid * (NS * _CPT) + sid * _CPT
    pltpu.sync_copy(dinv_hbm, dinvv)
    for j in range(_ROWS_PT // 128):
        pltpu.sync_copy(z128_hbm, degsp_acc.at[pl.ds(sid * _ROWS_PT + j * 128, 128)])
    plsc.subcore_barrier()

    it16 = _iota16()
    gbufs = (g0, g1)
    gsems = (gs0, gs1)
    ssems = (ss0, ss1)

    def super_body(s, carry):
        hb = base + s * _B8
        pltpu.sync_copy(src_hbm.at[pl.ds(hb, _B8)], srcv)
        pltpu.sync_copy(dst_hbm.at[pl.ds(hb, _B8)], dstv)
        pltpu.sync_copy(ew_hbm.at[pl.ds(hb, _B8)], ewv)

        gds = {0: pltpu.async_copy(h_hbm.at[srcv.at[0]], g0, gs0)}
        sds = {}
        for k in range(_B8):
            b = k % 2
            gds[k].wait()
            if k + 1 < _B8:
                nb = (k + 1) % 2
                if k >= 1:
                    sds[k - 1].wait()
                gds[k + 1] = pltpu.async_copy(
                    h_hbm.at[srcv.at[k + 1]], gbufs[nb], gsems[nb])
            f16s = []
            for g in range(8):
                ew16 = _clean_ew16(ewv[k, pl.ds(g * 16, 16)])
                src16 = srcv[k, pl.ds(g * 16, 16)]
                f16s.append(ew16 * plsc.load_gather(dinvv, [src16]))
            gb = gbufs[b]

            def feat_body(f, carry2):
                fv = jnp.full((16,), f, _i32)
                for g in range(8):
                    e16 = it16 + g * 16
                    v = plsc.load_gather(gb, [e16, fv])
                    plsc.store_scatter(gb, [e16, fv], v * f16s[g])
                return carry2

            lax.fori_loop(0, HID, feat_body, 0, unroll=4)
            sds[k] = pltpu.async_copy(
                gb, degsp_acc.at[dstv.at[k]], ssems[b], add=True)
        sds[_B8 - 2].wait()
        sds[_B8 - 1].wait()
        return carry

    lax.fori_loop(0, _CPT // _B8, super_body, 0)
    plsc.subcore_barrier()

    for j in range(_ROWS_PT // 128):
        r0 = sid * _ROWS_PT + j * 128
        pltpu.sync_copy(degsp_acc.at[pl.ds(r0, 128)], g0)
        pltpu.sync_copy(g0, out_hbm.at[cid, pl.ds(r0, 128)])


def _sc_layer(src2, dst3, ew2, dinv, h, z128):
    kfn = pl.kernel(
        _sc_layer_body,
        out_type=jax.ShapeDtypeStruct((NC, NP, HID), _f32),
        mesh=_sc_mesh(),
        compiler_params=pltpu.CompilerParams(needs_layout_passes=False),
        scratch_types=[
            pltpu.VMEM((_B8, CH), _i32),
            pltpu.VMEM((_B8, CH), _i32),
            pltpu.VMEM((_B8, CH), _f32),
            pltpu.VMEM((NP,), _f32),
            pltpu.VMEM((CH, HID), _f32),
            pltpu.VMEM((CH, HID), _f32),
            pltpu.SemaphoreType.DMA,
            pltpu.SemaphoreType.DMA,
            pltpu.SemaphoreType.DMA,
            pltpu.SemaphoreType.DMA,
            pltpu.VMEM_SHARED((NP, HID), _f32),
        ],
    )
    return kfn(src2, dst3, ew2, dinv, h, z128)


_RB = 128
_GRID = NP // _RB


def _tc_prep_body(comb_ref, bn0w_ref, bn0b_ref, w1t_ref, h1_ref):
    x = comb_ref[...]
    x = jnp.where(jnp.isnan(x), 0.0, x)
    x = jnp.clip(x, -1e6, 1e6)
    x = x * (bn0w_ref[...] * BN_SCALE) + bn0b_ref[...]
    x = jnp.clip(x, -10.0, 10.0)
    h1_ref[...] = jnp.dot(x, w1t_ref[...], preferred_element_type=_f32)


def _tc_prep(comb_p, bn0w, bn0b, w1t):
    return pl.pallas_call(
        _tc_prep_body,
        grid=(_GRID,),
        in_specs=[
            pl.BlockSpec((_RB, COMB), lambda i: (i, 0)),
            pl.BlockSpec((1, COMB), lambda i: (0, 0)),
            pl.BlockSpec((1, COMB), lambda i: (0, 0)),
            pl.BlockSpec((COMB, HID), lambda i: (0, 0)),
        ],
        out_specs=pl.BlockSpec((_RB, HID), lambda i: (i, 0)),
        out_shape=jax.ShapeDtypeStruct((NP, HID), _f32),
    )(comb_p, bn0w, bn0b, w1t)


def _tc_post_mid_body(accp_ref, h_ref, deg16_ref, b_ref, bnw_ref, bnb_ref,
                      wt_ref, hn_ref):
    acc = accp_ref[0] + accp_ref[1]
    deg = 1.0 + deg16_ref[...][:, 0:1]
    dinv = lax.rsqrt(deg)
    o = dinv * acc + (dinv * dinv) * h_ref[...] + b_ref[...]
    o = jnp.clip(o, -10.0, 10.0)
    o = o * (bnw_ref[...] * BN_SCALE) + bnb_ref[...]
    o = jnp.clip(o, -10.0, 10.0)
    o = jnp.maximum(o, 0.0)
    hn_ref[...] = jnp.dot(o, wt_ref[...], preferred_element_type=_f32)


def _tc_post_mid(accp, h, deg16, b, bnw, bnb, wt):
    return pl.pallas_call(
        _tc_post_mid_body,
        grid=(_GRID,),
        in_specs=[
            pl.BlockSpec((NC, _RB, HID), lambda i: (0, i, 0)),
            pl.BlockSpec((_RB, HID), lambda i: (i, 0)),
            pl.BlockSpec((_RB, 16), lambda i: (i, 0)),
            pl.BlockSpec((1, HID), lambda i: (0, 0)),
            pl.BlockSpec((1, HID), lambda i: (0, 0)),
            pl.BlockSpec((1, HID), lambda i: (0, 0)),
            pl.BlockSpec((HID, HID), lambda i: (0, 0)),
        ],
        out_specs=pl.BlockSpec((_RB, HID), lambda i: (i, 0)),
        out_shape=jax.ShapeDtypeStruct((NP, HID), _f32),
    )(accp, h, deg16, b, bnw, bnb, wt)


def _tc_post_final_body(accp_ref, h_ref, deg16_ref, b_ref, bnw_ref, bnb_ref,
                        linwt_ref, linb_ref, y_ref):
    acc = accp_ref[0] + accp_ref[1]
    deg = 1.0 + deg16_ref[...][:, 0:1]
    dinv = lax.rsqrt(deg)
    o = dinv * acc + (dinv * dinv) * h_ref[...] + b_ref[...]
    o = jnp.clip(o, -10.0, 10.0)
    o = o * (bnw_ref[...] * BN_SCALE) + bnb_ref[...]
    o = jnp.clip(o, -10.0, 10.0)
    o = jnp.maximum(o, 0.0)
    y = jnp.dot(o, linwt_ref[...], preferred_element_type=_f32) + linb_ref[0, 0]
    y = jnp.clip(y, -10.0, 10.0)
    y_ref[...] = jnp.broadcast_to(y, (_RB, HID))


def _tc_post_final(accp, h, deg16, b, bnw, bnb, linwt, linb):
    return pl.pallas_call(
        _tc_post_final_body,
        grid=(_GRID,),
        in_specs=[
            pl.BlockSpec((NC, _RB, HID), lambda i: (0, i, 0)),
            pl.BlockSpec((_RB, HID), lambda i: (i, 0)),
            pl.BlockSpec((_RB, 16), lambda i: (i, 0)),
            pl.BlockSpec((1, HID), lambda i: (0, 0)),
            pl.BlockSpec((1, HID), lambda i: (0, 0)),
            pl.BlockSpec((1, HID), lambda i: (0, 0)),
            pl.BlockSpec((HID, 1), lambda i: (0, 0)),
            pl.BlockSpec((1, 1), lambda i: (0, 0)),
        ],
        out_specs=pl.BlockSpec((_RB, HID), lambda i: (i, 0)),
        out_shape=jax.ShapeDtypeStruct((NP, HID), _f32),
    )(accp, h, deg16, b, bnw, bnb, linwt, linb)


def kernel(x_static, x_dynamic, edge_index, edge_weight, bn0_w, bn0_b,
           W1, b1, bn1_w, bn1_b, W2, b2, bn2_w, bn2_b, lin_W, lin_b):
    flat = x_dynamic.reshape(N, -1)
    comb = jnp.concatenate([x_static, flat], axis=1)
    comb_p = jnp.pad(comb, ((0, NP - N), (0, 0)))

    src = edge_index[0]
    dst = edge_index[1]
    npad = EP - E
    pad_idx = (jnp.arange(npad, dtype=_i32) % (NP - N)) + N
    src_p = jnp.concatenate([src, pad_idx]).reshape(NCHUNK, CH)
    dst_p = jnp.concatenate([dst, pad_idx]).reshape(NCHUNK, CH)
    ew_p = jnp.concatenate(
        [edge_weight, jnp.zeros((npad,), _f32)]).reshape(NCHUNK, CH)

    zv = jnp.zeros((_ROWS_PT,), _f32)
    z128 = jnp.zeros((128, 128), _f32)

    deg16_flat, dinv = _sc_deg(dst_p, ew_p, zv)
    deg16 = deg16_flat.reshape(NP, 16)
    h1 = _tc_prep(comb_p, bn0_w.reshape(1, -1), bn0_b.reshape(1, -1), W1.T)
    accp1 = _sc_layer(src_p, dst_p, ew_p, dinv, h1, z128)
    h2 = _tc_post_mid(accp1, h1, deg16, b1.reshape(1, -1),
                      bn1_w.reshape(1, -1), bn1_b.reshape(1, -1), W2.T)
    accp2 = _sc_layer(src_p, dst_p, ew_p, dinv, h2, z128)
    yb = _tc_post_final(accp2, h2, deg16, b2.reshape(1, -1),
                        bn2_w.reshape(1, -1), bn2_b.reshape(1, -1),
                        lin_W.T, lin_b.reshape(1, 1))
    return yb[:N, 0]


# speedup vs baseline: 16.5604x; 6.4407x over previous
import functools

import jax
import jax.numpy as jnp
from jax import lax
from jax.experimental import pallas as pl
from jax.experimental.pallas import tpu as pltpu
from jax.experimental.pallas import tpu_sc as plsc

N = 10000
NP = 10240
E = 320000
CH = 128
NCHUNK = 2560
EP = NCHUNK * CH
NC = 2
NS = 16
HID = 128
COMB = 128
BN_SCALE = 1.0 / (1.0 + 1e-5) ** 0.5

_i32 = jnp.int32
_f32 = jnp.float32


def _sc_mesh():
    return plsc.VectorSubcoreMesh(
        core_axis_name="c", subcore_axis_name="s", num_cores=NC, num_subcores=NS
    )


def _iota16():
    return lax.iota(_i32, 16)


def _clean_ew16(v):
    v = jnp.where(v != v, 0.0, v)
    return jnp.clip(v, 0.0, 1e6)


def _rsqrt16(x):
    i = lax.bitcast_convert_type(x, _i32)
    y = lax.bitcast_convert_type(jnp.int32(0x5F3759DF) - (i >> 1), _f32)
    for _ in range(3):
        y = y * (1.5 - 0.5 * x * y * y)
    return y


_DEG_CPT = NCHUNK // NS
_DEG_NPT = NP // (NC * NS)
_ROWS_PT = NP // NS


def _sc_deg_body(dst_hbm, ew_hbm, zv_hbm, deg16_hbm, dinv_hbm,
                 dstv, ewv, bounce, dinvbuf, b16, degsp):
    cid = lax.axis_index("c")
    sid = lax.axis_index("s")
    base = sid * _DEG_CPT
    pltpu.sync_copy(dst_hbm.at[pl.ds(base, _DEG_CPT)], dstv)
    pltpu.sync_copy(ew_hbm.at[pl.ds(base, _DEG_CPT)], ewv)
    pltpu.sync_copy(zv_hbm, degsp.at[pl.ds(sid * _ROWS_PT, _ROWS_PT)])
    plsc.subcore_barrier()

    it16 = _iota16()

    def clean_body(i, carry):
        c = i >> 3
        g = i & 7
        ewv[c, pl.ds(g * 16, 16)] = _clean_ew16(ewv[c, pl.ds(g * 16, 16)])
        return carry

    lax.fori_loop(0, _DEG_CPT * 8, clean_body, 0)

    def chunk_body(c, carry):
        pltpu.sync_copy(ewv.at[c], degsp.at[dstv.at[c]], add=True)
        return carry

    lax.fori_loop(0, _DEG_CPT, chunk_body, 0)
    plsc.subcore_barrier()

    node0 = cid * (NP // NC) + sid * _DEG_NPT
    pltpu.sync_copy(degsp.at[pl.ds(node0, _DEG_NPT)], bounce)

    def dinv_body(k, carry):
        d16 = bounce[pl.ds(k * 16, 16)]
        dinvbuf[pl.ds(k * 16, 16)] = _rsqrt16(1.0 + d16)
        plsc.store_scatter(b16, [(it16 + k * 16) * 16], d16)
        return carry

    lax.fori_loop(0, _DEG_NPT // 16, dinv_body, 0)
    pltpu.sync_copy(b16, deg16_hbm.at[pl.ds(node0 * 16, _DEG_NPT * 16)])
    pltpu.sync_copy(dinvbuf, dinv_hbm.at[pl.ds(node0, _DEG_NPT)])


def _sc_deg(dst2, ew2, zv):
    kfn = pl.kernel(
        _sc_deg_body,
        out_type=(
            jax.ShapeDtypeStruct((NP * 16,), _f32),
            jax.ShapeDtypeStruct((NP,), _f32),
        ),
        mesh=_sc_mesh(),
        compiler_params=pltpu.CompilerParams(needs_layout_passes=False),
        scratch_types=[
            pltpu.VMEM((_DEG_CPT, CH), _i32),
            pltpu.VMEM((_DEG_CPT, CH), _f32),
            pltpu.VMEM((_DEG_NPT,), _f32),
            pltpu.VMEM((_DEG_NPT,), _f32),
            pltpu.VMEM((_DEG_NPT * 16,), _f32),
            pltpu.VMEM_SHARED((NP,), _f32),
        ],
    )
    return kfn(dst2, ew2, zv)


_CPT = NCHUNK // (NC * NS)


_B8 = 8


def _sc_layer_body(src_hbm, dst_hbm, ew_hbm, dinv_hbm, h_hbm, z128_hbm,
                   out_hbm, srcv, dstv, ewv, dinvv, g0, g1,
                   gs0, gs1, ss0, ss1, degsp_acc):
    cid = lax.axis_index("c")
    sid = lax.axis_index("s")
    base = cid * (NS * _CPT) + sid * _CPT
    pltpu.sync_copy(dinv_hbm, dinvv)
    for j in range(_ROWS_PT // 128):
        pltpu.sync_copy(z128_hbm, degsp_acc.at[pl.ds(sid * _ROWS_PT + j * 128, 128)])
    plsc.subcore_barrier()

    it16 = _iota16()
    gbufs = (g0, g1)
    gsems = (gs0, gs1)
    ssems = (ss0, ss1)

    def super_body(s, carry):
        hb = base + s * _B8
        pltpu.sync_copy(src_hbm.at[pl.ds(hb, _B8)], srcv)
        pltpu.sync_copy(dst_hbm.at[pl.ds(hb, _B8)], dstv)
        pltpu.sync_copy(ew_hbm.at[pl.ds(hb, _B8)], ewv)

        gds = {0: pltpu.async_copy(h_hbm.at[srcv.at[0]], g0, gs0)}
        sds = {}
        for k in range(_B8):
            b = k % 2
            gds[k].wait()
            if k + 1 < _B8:
                nb = (k + 1) % 2
                if k >= 1:
                    sds[k - 1].wait()
                gds[k + 1] = pltpu.async_copy(
                    h_hbm.at[srcv.at[k + 1]], gbufs[nb], gsems[nb])
            gb = gbufs[b]
            for g in range(8):
                ew16 = _clean_ew16(ewv[k, pl.ds(g * 16, 16)])
                src16 = srcv[k, pl.ds(g * 16, 16)]
                f16 = ew16 * plsc.load_gather(dinvv, [src16])

                def row_body(l, carry2, g=g, f16=f16):
                    bc = jnp.take_along_axis(f16, jnp.full((16,), l, _i32),
                                             axis=0)
                    e = g * 16 + l
                    for j in range(8):
                        sl = pl.ds(j * 16, 16)
                        gb[e, sl] = gb[e, sl] * bc
                    return carry2

                lax.fori_loop(0, 16, row_body, 0)
            sds[k] = pltpu.async_copy(
                gb, degsp_acc.at[dstv.at[k]], ssems[b], add=True)
        sds[_B8 - 2].wait()
        sds[_B8 - 1].wait()
        return carry

    lax.fori_loop(0, _CPT // _B8, super_body, 0)
    plsc.subcore_barrier()

    for j in range(_ROWS_PT // 128):
        r0 = sid * _ROWS_PT + j * 128
        pltpu.sync_copy(degsp_acc.at[pl.ds(r0, 128)], g0)
        pltpu.sync_copy(g0, out_hbm.at[cid, pl.ds(r0, 128)])


def _sc_layer(src2, dst3, ew2, dinv, h, z128):
    kfn = pl.kernel(
        _sc_layer_body,
        out_type=jax.ShapeDtypeStruct((NC, NP, HID), _f32),
        mesh=_sc_mesh(),
        compiler_params=pltpu.CompilerParams(needs_layout_passes=False),
        scratch_types=[
            pltpu.VMEM((_B8, CH), _i32),
            pltpu.VMEM((_B8, CH), _i32),
            pltpu.VMEM((_B8, CH), _f32),
            pltpu.VMEM((NP,), _f32),
            pltpu.VMEM((CH, HID), _f32),
            pltpu.VMEM((CH, HID), _f32),
            pltpu.SemaphoreType.DMA,
            pltpu.SemaphoreType.DMA,
            pltpu.SemaphoreType.DMA,
            pltpu.SemaphoreType.DMA,
            pltpu.VMEM_SHARED((NP, HID), _f32),
        ],
    )
    return kfn(src2, dst3, ew2, dinv, h, z128)


_RB = 128
_GRID = NP // _RB


def _tc_prep_body(comb_ref, bn0w_ref, bn0b_ref, w1t_ref, h1_ref):
    x = comb_ref[...]
    x = jnp.where(jnp.isnan(x), 0.0, x)
    x = jnp.clip(x, -1e6, 1e6)
    x = x * (bn0w_ref[...] * BN_SCALE) + bn0b_ref[...]
    x = jnp.clip(x, -10.0, 10.0)
    h1_ref[...] = jnp.dot(x, w1t_ref[...], preferred_element_type=_f32)


def _tc_prep(comb_p, bn0w, bn0b, w1t):
    return pl.pallas_call(
        _tc_prep_body,
        grid=(_GRID,),
        in_specs=[
            pl.BlockSpec((_RB, COMB), lambda i: (i, 0)),
            pl.BlockSpec((1, COMB), lambda i: (0, 0)),
            pl.BlockSpec((1, COMB), lambda i: (0, 0)),
            pl.BlockSpec((COMB, HID), lambda i: (0, 0)),
        ],
        out_specs=pl.BlockSpec((_RB, HID), lambda i: (i, 0)),
        out_shape=jax.ShapeDtypeStruct((NP, HID), _f32),
    )(comb_p, bn0w, bn0b, w1t)


def _tc_post_mid_body(accp_ref, h_ref, deg16_ref, b_ref, bnw_ref, bnb_ref,
                      wt_ref, hn_ref):
    acc = accp_ref[0] + accp_ref[1]
    deg = 1.0 + deg16_ref[...][:, 0:1]
    dinv = lax.rsqrt(deg)
    o = dinv * acc + (dinv * dinv) * h_ref[...] + b_ref[...]
    o = jnp.clip(o, -10.0, 10.0)
    o = o * (bnw_ref[...] * BN_SCALE) + bnb_ref[...]
    o = jnp.clip(o, -10.0, 10.0)
    o = jnp.maximum(o, 0.0)
    hn_ref[...] = jnp.dot(o, wt_ref[...], preferred_element_type=_f32)


def _tc_post_mid(accp, h, deg16, b, bnw, bnb, wt):
    return pl.pallas_call(
        _tc_post_mid_body,
        grid=(_GRID,),
        in_specs=[
            pl.BlockSpec((NC, _RB, HID), lambda i: (0, i, 0)),
            pl.BlockSpec((_RB, HID), lambda i: (i, 0)),
            pl.BlockSpec((_RB, 16), lambda i: (i, 0)),
            pl.BlockSpec((1, HID), lambda i: (0, 0)),
            pl.BlockSpec((1, HID), lambda i: (0, 0)),
            pl.BlockSpec((1, HID), lambda i: (0, 0)),
            pl.BlockSpec((HID, HID), lambda i: (0, 0)),
        ],
        out_specs=pl.BlockSpec((_RB, HID), lambda i: (i, 0)),
        out_shape=jax.ShapeDtypeStruct((NP, HID), _f32),
    )(accp, h, deg16, b, bnw, bnb, wt)


def _tc_post_final_body(accp_ref, h_ref, deg16_ref, b_ref, bnw_ref, bnb_ref,
                        linwt_ref, linb_ref, y_ref):
    acc = accp_ref[0] + accp_ref[1]
    deg = 1.0 + deg16_ref[...][:, 0:1]
    dinv = lax.rsqrt(deg)
    o = dinv * acc + (dinv * dinv) * h_ref[...] + b_ref[...]
    o = jnp.clip(o, -10.0, 10.0)
    o = o * (bnw_ref[...] * BN_SCALE) + bnb_ref[...]
    o = jnp.clip(o, -10.0, 10.0)
    o = jnp.maximum(o, 0.0)
    y = jnp.dot(o, linwt_ref[...], preferred_element_type=_f32) + linb_ref[0, 0]
    y = jnp.clip(y, -10.0, 10.0)
    y_ref[...] = jnp.broadcast_to(y, (_RB, HID))


def _tc_post_final(accp, h, deg16, b, bnw, bnb, linwt, linb):
    return pl.pallas_call(
        _tc_post_final_body,
        grid=(_GRID,),
        in_specs=[
            pl.BlockSpec((NC, _RB, HID), lambda i: (0, i, 0)),
            pl.BlockSpec((_RB, HID), lambda i: (i, 0)),
            pl.BlockSpec((_RB, 16), lambda i: (i, 0)),
            pl.BlockSpec((1, HID), lambda i: (0, 0)),
            pl.BlockSpec((1, HID), lambda i: (0, 0)),
            pl.BlockSpec((1, HID), lambda i: (0, 0)),
            pl.BlockSpec((HID, 1), lambda i: (0, 0)),
            pl.BlockSpec((1, 1), lambda i: (0, 0)),
        ],
        out_specs=pl.BlockSpec((_RB, HID), lambda i: (i, 0)),
        out_shape=jax.ShapeDtypeStruct((NP, HID), _f32),
    )(accp, h, deg16, b, bnw, bnb, linwt, linb)


def kernel(x_static, x_dynamic, edge_index, edge_weight, bn0_w, bn0_b,
           W1, b1, bn1_w, bn1_b, W2, b2, bn2_w, bn2_b, lin_W, lin_b):
    flat = x_dynamic.reshape(N, -1)
    comb = jnp.concatenate([x_static, flat], axis=1)
    comb_p = jnp.pad(comb, ((0, NP - N), (0, 0)))

    src = edge_index[0]
    dst = edge_index[1]
    npad = EP - E
    pad_idx = (jnp.arange(npad, dtype=_i32) % (NP - N)) + N
    src_p = jnp.concatenate([src, pad_idx]).reshape(NCHUNK, CH)
    dst_p = jnp.concatenate([dst, pad_idx]).reshape(NCHUNK, CH)
    ew_p = jnp.concatenate(
        [edge_weight, jnp.zeros((npad,), _f32)]).reshape(NCHUNK, CH)

    zv = jnp.zeros((_ROWS_PT,), _f32)
    z128 = jnp.zeros((128, 128), _f32)

    deg16_flat, dinv = _sc_deg(dst_p, ew_p, zv)
    deg16 = deg16_flat.reshape(NP, 16)
    h1 = _tc_prep(comb_p, bn0_w.reshape(1, -1), bn0_b.reshape(1, -1), W1.T)
    accp1 = _sc_layer(src_p, dst_p, ew_p, dinv, h1, z128)
    h2 = _tc_post_mid(accp1, h1, deg16, b1.reshape(1, -1),
                      bn1_w.reshape(1, -1), bn1_b.reshape(1, -1), W2.T)
    accp2 = _sc_layer(src_p, dst_p, ew_p, dinv, h2, z128)
    yb = _tc_post_final(accp2, h2, deg16, b2.reshape(1, -1),
                        bn2_w.reshape(1, -1), bn2_b.reshape(1, -1),
                        lin_W.T, lin_b.reshape(1, 1))
    return yb[:N, 0]


# speedup vs baseline: 20.7913x; 1.2555x over previous
import functools

import jax
import jax.numpy as jnp
from jax import lax
from jax.experimental import pallas as pl
from jax.experimental.pallas import tpu as pltpu
from jax.experimental.pallas import tpu_sc as plsc

N = 10000
NP = 10240
E = 320000
CH = 128
NCHUNK = 2560
EP = NCHUNK * CH
NC = 2
NS = 16
HID = 128
COMB = 128
BN_SCALE = 1.0 / (1.0 + 1e-5) ** 0.5

_i32 = jnp.int32
_f32 = jnp.float32


def _sc_mesh():
    return plsc.VectorSubcoreMesh(
        core_axis_name="c", subcore_axis_name="s", num_cores=NC, num_subcores=NS
    )


def _iota16():
    return lax.iota(_i32, 16)


def _clean_ew16(v):
    v = jnp.where(v != v, 0.0, v)
    return jnp.clip(v, 0.0, 1e6)


def _rsqrt16(x):
    i = lax.bitcast_convert_type(x, _i32)
    y = lax.bitcast_convert_type(jnp.int32(0x5F3759DF) - (i >> 1), _f32)
    for _ in range(3):
        y = y * (1.5 - 0.5 * x * y * y)
    return y


_DEG_CPT = NCHUNK // NS
_DEG_NPT = NP // (NC * NS)
_ROWS_PT = NP // NS


def _sc_deg_body(dst_hbm, ew_hbm, deg16_hbm, dinv_hbm,
                 dstv, ewv, bounce, dinvbuf, b16, degsp):
    cid = lax.axis_index("c")
    sid = lax.axis_index("s")
    base = sid * _DEG_CPT
    pltpu.sync_copy(dst_hbm.at[pl.ds(base, _DEG_CPT)], dstv)
    pltpu.sync_copy(ew_hbm.at[pl.ds(base, _DEG_CPT)], ewv)

    def zmem_body(r, carry):
        dinvbuf[pl.ds(r * 16, 16)] = jnp.zeros((16,), _f32)
        return carry

    lax.fori_loop(0, _DEG_NPT // 16, zmem_body, 0)
    pltpu.sync_copy(dinvbuf, degsp.at[pl.ds(sid * _ROWS_PT, _DEG_NPT)])
    pltpu.sync_copy(dinvbuf, degsp.at[pl.ds(sid * _ROWS_PT + _DEG_NPT, _DEG_NPT)])
    plsc.subcore_barrier()

    it16 = _iota16()

    def clean_body(i, carry):
        c = i >> 3
        g = i & 7
        ewv[c, pl.ds(g * 16, 16)] = _clean_ew16(ewv[c, pl.ds(g * 16, 16)])
        return carry

    lax.fori_loop(0, _DEG_CPT * 8, clean_body, 0)

    def chunk_body(c, carry):
        pltpu.sync_copy(ewv.at[c], degsp.at[dstv.at[c]], add=True)
        return carry

    lax.fori_loop(0, _DEG_CPT, chunk_body, 0)
    plsc.subcore_barrier()

    node0 = cid * (NP // NC) + sid * _DEG_NPT
    pltpu.sync_copy(degsp.at[pl.ds(node0, _DEG_NPT)], bounce)

    def dinv_body(k, carry):
        d16 = bounce[pl.ds(k * 16, 16)]
        dinvbuf[pl.ds(k * 16, 16)] = _rsqrt16(1.0 + d16)
        plsc.store_scatter(b16, [(it16 + k * 16) * 16], d16)
        return carry

    lax.fori_loop(0, _DEG_NPT // 16, dinv_body, 0)
    pltpu.sync_copy(b16, deg16_hbm.at[pl.ds(node0 * 16, _DEG_NPT * 16)])
    pltpu.sync_copy(dinvbuf, dinv_hbm.at[pl.ds(node0, _DEG_NPT)])


def _sc_deg(dst2, ew2):
    kfn = pl.kernel(
        _sc_deg_body,
        out_type=(
            jax.ShapeDtypeStruct((NP * 16,), _f32),
            jax.ShapeDtypeStruct((NP,), _f32),
        ),
        mesh=_sc_mesh(),
        compiler_params=pltpu.CompilerParams(needs_layout_passes=False),
        scratch_types=[
            pltpu.VMEM((_DEG_CPT, CH), _i32),
            pltpu.VMEM((_DEG_CPT, CH), _f32),
            pltpu.VMEM((_DEG_NPT,), _f32),
            pltpu.VMEM((_DEG_NPT,), _f32),
            pltpu.VMEM((_DEG_NPT * 16,), _f32),
            pltpu.VMEM_SHARED((NP,), _f32),
        ],
    )
    return kfn(dst2, ew2)


_CPT = NCHUNK // (NC * NS)


_B8 = 8


def _sc_layer_body(src_hbm, dst_hbm, ew_hbm, dinv_hbm, h_hbm,
                   out_hbm, srcv, dstv, ewv, dinvv, g0, g1,
                   gs0, gs1, ss0, ss1, degsp_acc):
    cid = lax.axis_index("c")
    sid = lax.axis_index("s")
    base = cid * (NS * _CPT) + sid * _CPT
    pltpu.sync_copy(dinv_hbm, dinvv)

    def zmem_body(r, carry):
        for j in range(8):
            g0[r, pl.ds(j * 16, 16)] = jnp.zeros((16,), _f32)
        return carry

    lax.fori_loop(0, CH, zmem_body, 0)
    for j in range(_ROWS_PT // 128):
        pltpu.sync_copy(g0, degsp_acc.at[pl.ds(sid * _ROWS_PT + j * 128, 128)])
    plsc.subcore_barrier()

    it16 = _iota16()
    gbufs = (g0, g1)
    gsems = (gs0, gs1)
    ssems = (ss0, ss1)

    def super_body(s, carry):
        hb = base + s * _B8
        pltpu.sync_copy(src_hbm.at[pl.ds(hb, _B8)], srcv)
        pltpu.sync_copy(dst_hbm.at[pl.ds(hb, _B8)], dstv)
        pltpu.sync_copy(ew_hbm.at[pl.ds(hb, _B8)], ewv)

        gds = {0: pltpu.async_copy(h_hbm.at[srcv.at[0]], g0, gs0)}
        sds = {}
        for k in range(_B8):
            b = k % 2
            gds[k].wait()
            if k + 1 < _B8:
                nb = (k + 1) % 2
                if k >= 1:
                    sds[k - 1].wait()
                gds[k + 1] = pltpu.async_copy(
                    h_hbm.at[srcv.at[k + 1]], gbufs[nb], gsems[nb])
            gb = gbufs[b]
            for g in range(8):
                ew16 = _clean_ew16(ewv[k, pl.ds(g * 16, 16)])
                src16 = srcv[k, pl.ds(g * 16, 16)]
                f16 = ew16 * plsc.load_gather(dinvv, [src16])

                def row_body(l, carry2, g=g, f16=f16):
                    bc = jnp.take_along_axis(f16, jnp.full((16,), l, _i32),
                                             axis=0)
                    e = g * 16 + l
                    for j in range(8):
                        sl = pl.ds(j * 16, 16)
                        gb[e, sl] = gb[e, sl] * bc
                    return carry2

                lax.fori_loop(0, 16, row_body, 0)
            sds[k] = pltpu.async_copy(
                gb, degsp_acc.at[dstv.at[k]], ssems[b], add=True)
        sds[_B8 - 2].wait()
        sds[_B8 - 1].wait()
        return carry

    lax.fori_loop(0, _CPT // _B8, super_body, 0)
    plsc.subcore_barrier()

    for j in range(_ROWS_PT // 128):
        r0 = sid * _ROWS_PT + j * 128
        pltpu.sync_copy(degsp_acc.at[pl.ds(r0, 128)], g0)
        pltpu.sync_copy(g0, out_hbm.at[cid, pl.ds(r0, 128)])


def _sc_layer(src2, dst3, ew2, dinv, h):
    kfn = pl.kernel(
        _sc_layer_body,
        out_type=jax.ShapeDtypeStruct((NC, NP, HID), _f32),
        mesh=_sc_mesh(),
        compiler_params=pltpu.CompilerParams(needs_layout_passes=False),
        scratch_types=[
            pltpu.VMEM((_B8, CH), _i32),
            pltpu.VMEM((_B8, CH), _i32),
            pltpu.VMEM((_B8, CH), _f32),
            pltpu.VMEM((NP,), _f32),
            pltpu.VMEM((CH, HID), _f32),
            pltpu.VMEM((CH, HID), _f32),
            pltpu.SemaphoreType.DMA,
            pltpu.SemaphoreType.DMA,
            pltpu.SemaphoreType.DMA,
            pltpu.SemaphoreType.DMA,
            pltpu.VMEM_SHARED((NP, HID), _f32),
        ],
    )
    return kfn(src2, dst3, ew2, dinv, h)


_RB = 512
_GRID = NP // _RB


def _tc_prep_body(comb_ref, bn0w_ref, bn0b_ref, w1t_ref, h1_ref):
    x = comb_ref[...]
    x = jnp.where(jnp.isnan(x), 0.0, x)
    x = jnp.clip(x, -1e6, 1e6)
    x = x * (bn0w_ref[...] * BN_SCALE) + bn0b_ref[...]
    x = jnp.clip(x, -10.0, 10.0)
    h1_ref[...] = jnp.dot(x, w1t_ref[...], preferred_element_type=_f32)


def _tc_prep(comb_p, bn0w, bn0b, w1t):
    return pl.pallas_call(
        _tc_prep_body,
        grid=(_GRID,),
        in_specs=[
            pl.BlockSpec((_RB, COMB), lambda i: (i, 0)),
            pl.BlockSpec((1, COMB), lambda i: (0, 0)),
            pl.BlockSpec((1, COMB), lambda i: (0, 0)),
            pl.BlockSpec((COMB, HID), lambda i: (0, 0)),
        ],
        out_specs=pl.BlockSpec((_RB, HID), lambda i: (i, 0)),
        out_shape=jax.ShapeDtypeStruct((NP, HID), _f32),
    )(comb_p, bn0w, bn0b, w1t)


def _tc_post_mid_body(accp_ref, h_ref, deg16_ref, b_ref, bnw_ref, bnb_ref,
                      wt_ref, hn_ref):
    acc = accp_ref[0] + accp_ref[1]
    deg = 1.0 + deg16_ref[...][:, 0:1]
    dinv = lax.rsqrt(deg)
    o = dinv * acc + (dinv * dinv) * h_ref[...] + b_ref[...]
    o = jnp.clip(o, -10.0, 10.0)
    o = o * (bnw_ref[...] * BN_SCALE) + bnb_ref[...]
    o = jnp.clip(o, -10.0, 10.0)
    o = jnp.maximum(o, 0.0)
    hn_ref[...] = jnp.dot(o, wt_ref[...], preferred_element_type=_f32)


def _tc_post_mid(accp, h, deg16, b, bnw, bnb, wt):
    return pl.pallas_call(
        _tc_post_mid_body,
        grid=(_GRID,),
        in_specs=[
            pl.BlockSpec((NC, _RB, HID), lambda i: (0, i, 0)),
            pl.BlockSpec((_RB, HID), lambda i: (i, 0)),
            pl.BlockSpec((_RB, 16), lambda i: (i, 0)),
            pl.BlockSpec((1, HID), lambda i: (0, 0)),
            pl.BlockSpec((1, HID), lambda i: (0, 0)),
            pl.BlockSpec((1, HID), lambda i: (0, 0)),
            pl.BlockSpec((HID, HID), lambda i: (0, 0)),
        ],
        out_specs=pl.BlockSpec((_RB, HID), lambda i: (i, 0)),
        out_shape=jax.ShapeDtypeStruct((NP, HID), _f32),
    )(accp, h, deg16, b, bnw, bnb, wt)


def _tc_post_final_body(accp_ref, h_ref, deg16_ref, b_ref, bnw_ref, bnb_ref,
                        linwt_ref, linb_ref, y_ref):
    acc = accp_ref[0] + accp_ref[1]
    deg = 1.0 + deg16_ref[...][:, 0:1]
    dinv = lax.rsqrt(deg)
    o = dinv * acc + (dinv * dinv) * h_ref[...] + b_ref[...]
    o = jnp.clip(o, -10.0, 10.0)
    o = o * (bnw_ref[...] * BN_SCALE) + bnb_ref[...]
    o = jnp.clip(o, -10.0, 10.0)
    o = jnp.maximum(o, 0.0)
    y = jnp.dot(o, linwt_ref[...], preferred_element_type=_f32) + linb_ref[0, 0]
    y = jnp.clip(y, -10.0, 10.0)
    y_ref[...] = jnp.broadcast_to(y, (_RB, HID))


def _tc_post_final(accp, h, deg16, b, bnw, bnb, linwt, linb):
    return pl.pallas_call(
        _tc_post_final_body,
        grid=(_GRID,),
        in_specs=[
            pl.BlockSpec((NC, _RB, HID), lambda i: (0, i, 0)),
            pl.BlockSpec((_RB, HID), lambda i: (i, 0)),
            pl.BlockSpec((_RB, 16), lambda i: (i, 0)),
            pl.BlockSpec((1, HID), lambda i: (0, 0)),
            pl.BlockSpec((1, HID), lambda i: (0, 0)),
            pl.BlockSpec((1, HID), lambda i: (0, 0)),
            pl.BlockSpec((HID, 1), lambda i: (0, 0)),
            pl.BlockSpec((1, 1), lambda i: (0, 0)),
        ],
        out_specs=pl.BlockSpec((_RB, HID), lambda i: (i, 0)),
        out_shape=jax.ShapeDtypeStruct((NP, HID), _f32),
    )(accp, h, deg16, b, bnw, bnb, linwt, linb)


def kernel(x_static, x_dynamic, edge_index, edge_weight, bn0_w, bn0_b,
           W1, b1, bn1_w, bn1_b, W2, b2, bn2_w, bn2_b, lin_W, lin_b):
    flat = x_dynamic.reshape(N, -1)
    comb = jnp.concatenate([x_static, flat], axis=1)
    comb_p = jnp.pad(comb, ((0, NP - N), (0, 0)))

    src = edge_index[0]
    dst = edge_index[1]
    npad = EP - E
    pad_idx = (jnp.arange(npad, dtype=_i32) % (NP - N)) + N
    src_p = jnp.concatenate([src, pad_idx]).reshape(NCHUNK, CH)
    dst_p = jnp.concatenate([dst, pad_idx]).reshape(NCHUNK, CH)
    ew_p = jnp.concatenate(
        [edge_weight, jnp.zeros((npad,), _f32)]).reshape(NCHUNK, CH)

    deg16_flat, dinv = _sc_deg(dst_p, ew_p)
    deg16 = deg16_flat.reshape(NP, 16)
    h1 = _tc_prep(comb_p, bn0_w.reshape(1, -1), bn0_b.reshape(1, -1), W1.T)
    accp1 = _sc_layer(src_p, dst_p, ew_p, dinv, h1)
    h2 = _tc_post_mid(accp1, h1, deg16, b1.reshape(1, -1),
                      bn1_w.reshape(1, -1), bn1_b.reshape(1, -1), W2.T)
    accp2 = _sc_layer(src_p, dst_p, ew_p, dinv, h2)
    yb = _tc_post_final(accp2, h2, deg16, b2.reshape(1, -1),
                        bn2_w.reshape(1, -1), bn2_b.reshape(1, -1),
                        lin_W.T, lin_b.reshape(1, 1))
    return yb[:N, 0]
